# Initial kernel scaffold; baseline (speedup 1.0000x reference)
import jax
import jax.numpy as jnp
from jax.experimental import pallas as pl


def kernel(x, edge_index, batch, W1, b1, W2, b2):
    raise NotImplementedError("write your pallas kernel here")


# speedup vs baseline: 19.4759x; 19.4759x over previous
import functools

import jax
import jax.numpy as jnp
from jax import lax
from jax.experimental import pallas as pl
from jax.experimental.pallas import tpu as pltpu
from jax.experimental.pallas import tpu_sc as plsc

_N = 10000
_E = 320000
_D = 128
_H = 128
_G = 64

_NC = 2
_NS = 16
_NW = _NC * _NS
_EPW = _E // _NW
_CH = 80
_NCHUNK = _EPW // _CH
_RPT = 624
_TAIL = _N - _NS * _RPT
_TAIL0 = _NS * _RPT

_RB = 1000
_NB = _N // _RB

_mesh = plsc.VectorSubcoreMesh(core_axis_name="c", subcore_axis_name="s")


@functools.partial(
    pl.kernel,
    mesh=_mesh,
    out_type=jax.ShapeDtypeStruct((2 * _N,), jnp.float32),
    scratch_types=[
        pltpu.VMEM((_NCHUNK, _CH), jnp.int32),
        pltpu.VMEM((_CH,), jnp.float32),
        pltpu.VMEM((_N,), jnp.float32),
        pltpu.VMEM_SHARED((_N,), jnp.float32),
    ],
)
def _deg_call(dst_hbm, deg_hbm, dst2d, ones_v, zero_v, deg_sh):
    cid = lax.axis_index("c")
    sid = lax.axis_index("s")
    wid = sid * _NC + cid
    pltpu.sync_copy(dst_hbm.at[wid], dst2d)

    def _fill_ones(i, c):
        ones_v[pl.ds(i * 16, 16)] = jnp.ones((16,), jnp.float32)
        return c

    lax.fori_loop(0, _CH // 16, _fill_ones, 0)

    @pl.when(sid == 0)
    def _():
        def _fill_zero(i, c):
            zero_v[pl.ds(i * 16, 16)] = jnp.zeros((16,), jnp.float32)
            return c

        lax.fori_loop(0, _N // 16, _fill_zero, 0)
        pltpu.sync_copy(zero_v, deg_sh)

    plsc.subcore_barrier()

    def _scat(j, c):
        pltpu.sync_copy(ones_v, deg_sh.at[dst2d.at[j]], add=True)
        return c

    lax.fori_loop(0, _NCHUNK, _scat, 0)
    plsc.subcore_barrier()

    @pl.when(sid == 0)
    def _():
        pltpu.sync_copy(deg_sh, zero_v)
        pltpu.sync_copy(zero_v, deg_hbm.at[pl.ds(cid * _N, _N)])


@functools.partial(
    pl.kernel,
    mesh=_mesh,
    out_type=jax.ShapeDtypeStruct((2 * _N, _H), jnp.float32),
    scratch_types=[
        pltpu.VMEM((_NCHUNK, _CH), jnp.int32),
        pltpu.VMEM((_NCHUNK, _CH), jnp.int32),
        pltpu.VMEM((_CH, _H), jnp.float32),
        pltpu.VMEM_SHARED((_N, _H), jnp.float32),
        pltpu.SemaphoreType.DMA,
    ],
)
def _agg_call(h_hbm, src_hbm, dst_hbm, zeros_hbm, out_hbm, src2d, dst2d,
              rows_v, acc_sh, sem):
    cid = lax.axis_index("c")
    sid = lax.axis_index("s")
    wid = sid * _NC + cid
    pltpu.sync_copy(src_hbm.at[wid], src2d)
    pltpu.sync_copy(dst_hbm.at[wid], dst2d)

    pltpu.sync_copy(zeros_hbm.at[pl.ds(sid * _RPT, _RPT)],
                    acc_sh.at[pl.ds(sid * _RPT, _RPT)])

    @pl.when(sid == 0)
    def _():
        pltpu.sync_copy(zeros_hbm.at[pl.ds(_TAIL0, _TAIL)],
                        acc_sh.at[pl.ds(_TAIL0, _TAIL)])

    plsc.subcore_barrier()

    def _step(j, c):
        pltpu.async_copy(h_hbm.at[src2d.at[j]], rows_v, sem).wait()
        pltpu.sync_copy(rows_v, acc_sh.at[dst2d.at[j]], add=True)
        return c

    lax.fori_loop(0, _NCHUNK, _step, 0)
    plsc.subcore_barrier()

    pltpu.sync_copy(acc_sh.at[pl.ds(sid * _RPT, _RPT)],
                    out_hbm.at[pl.ds(cid * _N + sid * _RPT, _RPT)])

    @pl.when(sid == 0)
    def _():
        pltpu.sync_copy(acc_sh.at[pl.ds(_TAIL0, _TAIL)],
                        out_hbm.at[pl.ds(cid * _N + _TAIL0, _TAIL)])


def _l1_body(x_ref, w_ref, d0_ref, d1_ref, hp_ref, s_ref):
    s = lax.rsqrt(1.0 + d0_ref[0] + d1_ref[0])
    hp_ref[...] = jnp.dot(x_ref[...], w_ref[...],
                          preferred_element_type=jnp.float32) * s
    s_ref[...] = s


def _l2_body(p0_ref, p1_ref, hp1_ref, s_ref, b1_ref, w_ref, hp2_ref):
    s = s_ref[...]
    z1 = jnp.maximum(
        s * (p0_ref[0] + p1_ref[0] + hp1_ref[...]) + b1_ref[...], 0.0)
    hp2_ref[...] = jnp.dot(z1, w_ref[...],
                           preferred_element_type=jnp.float32) * s


def _pool_body(q0_ref, q1_ref, hp2_ref, s_ref, b2_ref, bat_ref, out_ref,
               acc_ref, cnt_ref):
    i = pl.program_id(0)

    @pl.when(i == 0)
    def _():
        acc_ref[...] = jnp.zeros_like(acc_ref)
        cnt_ref[...] = jnp.zeros_like(cnt_ref)

    z2 = s_ref[...] * (q0_ref[0] + q1_ref[0] + hp2_ref[...]) + b2_ref[...]
    gid = lax.broadcasted_iota(jnp.int32, (_RB, _G), 1).astype(jnp.float32)
    p = (bat_ref[...] == gid).astype(jnp.float32)
    acc_ref[...] += lax.dot_general(p, z2, (((0,), (0,)), ((), ())),
                                    preferred_element_type=jnp.float32)
    cnt_ref[...] += lax.dot_general(p, jnp.ones((_RB, 1), jnp.float32),
                                    (((0,), (0,)), ((), ())),
                                    preferred_element_type=jnp.float32)

    @pl.when(i == _NB - 1)
    def _():
        out_ref[...] = acc_ref[...] / jnp.maximum(cnt_ref[...], 1.0)


def kernel(x, edge_index, batch, W1, b1, W2, b2):
    src3d = edge_index[0].reshape(_NW, _NCHUNK, _CH)
    dst3d = edge_index[1].reshape(_NW, _NCHUNK, _CH)
    zeros = jnp.zeros((_N, _H), jnp.float32)

    degp = _deg_call(dst3d).reshape(2, _N, 1)

    hp1, s = pl.pallas_call(
        _l1_body,
        grid=(_NB,),
        in_specs=[
            pl.BlockSpec((_RB, _D), lambda i: (i, 0)),
            pl.BlockSpec((_D, _H), lambda i: (0, 0)),
            pl.BlockSpec((1, _RB, 1), lambda i: (0, i, 0)),
            pl.BlockSpec((1, _RB, 1), lambda i: (1, i, 0)),
        ],
        out_specs=[
            pl.BlockSpec((_RB, _H), lambda i: (i, 0)),
            pl.BlockSpec((_RB, 1), lambda i: (i, 0)),
        ],
        out_shape=[
            jax.ShapeDtypeStruct((_N, _H), jnp.float32),
            jax.ShapeDtypeStruct((_N, 1), jnp.float32),
        ],
    )(x, W1, degp, degp)

    agg1 = _agg_call(hp1, src3d, dst3d, zeros).reshape(2, _N, _H)

    hp2 = pl.pallas_call(
        _l2_body,
        grid=(_NB,),
        in_specs=[
            pl.BlockSpec((1, _RB, _H), lambda i: (0, i, 0)),
            pl.BlockSpec((1, _RB, _H), lambda i: (1, i, 0)),
            pl.BlockSpec((_RB, _H), lambda i: (i, 0)),
            pl.BlockSpec((_RB, 1), lambda i: (i, 0)),
            pl.BlockSpec((1, _H), lambda i: (0, 0)),
            pl.BlockSpec((_H, _H), lambda i: (0, 0)),
        ],
        out_specs=pl.BlockSpec((_RB, _H), lambda i: (i, 0)),
        out_shape=jax.ShapeDtypeStruct((_N, _H), jnp.float32),
    )(agg1, agg1, hp1, s, b1.reshape(1, _H), W2)

    agg2 = _agg_call(hp2, src3d, dst3d, zeros).reshape(2, _N, _H)

    batf = batch.astype(jnp.float32).reshape(_N, 1)
    out = pl.pallas_call(
        _pool_body,
        grid=(_NB,),
        in_specs=[
            pl.BlockSpec((1, _RB, _H), lambda i: (0, i, 0)),
            pl.BlockSpec((1, _RB, _H), lambda i: (1, i, 0)),
            pl.BlockSpec((_RB, _H), lambda i: (i, 0)),
            pl.BlockSpec((_RB, 1), lambda i: (i, 0)),
            pl.BlockSpec((1, _H), lambda i: (0, 0)),
            pl.BlockSpec((_RB, 1), lambda i: (i, 0)),
        ],
        out_specs=pl.BlockSpec((_G, _H), lambda i: (0, 0)),
        out_shape=jax.ShapeDtypeStruct((_G, _H), jnp.float32),
        scratch_shapes=[
            pltpu.VMEM((_G, _H), jnp.float32),
            pltpu.VMEM((_G, 1), jnp.float32),
        ],
    )(agg2, agg2, hp2, s, b2.reshape(1, _H), batf)
    return out
